# baseline (speedup 1.0000x reference)
import functools

import jax
import jax.numpy as jnp
from jax import lax
from jax.experimental import pallas as pl
from jax.experimental.pallas import tpu as pltpu
from jax.experimental.pallas import tpu_sc as plsc

_K = 3
_L = 4
_B = 8
_V = 1000
_FULL = _V // 16
_TAIL = _V - 16
_NEW0 = _FULL * 16 - _TAIL
_EPS = 2.220446049250313e-16
_NEG_INF = float("-inf")
_BIG = 1 << 20
_SQRT2 = 1.4142135381698608
_LN2_HI = 0.693359375
_LN2_LO = -2.12194440054690583e-4


def _poly_log(x):
    bits = lax.bitcast_convert_type(x, jnp.int32)
    e = (bits >> 23) - 127
    m = lax.bitcast_convert_type(
        (bits & 0x7FFFFF) | 0x3F800000, jnp.float32)
    big = m > _SQRT2
    m = jnp.where(big, m * 0.5, m)
    e = jnp.where(big, e + 1, e)
    r = (m - 1.0) / (m + 1.0)
    t = r * r
    poly = 1.0 + t * (1.0 / 3.0 + t * (0.2 + t * (1.0 / 7.0 + t * (1.0 / 9.0))))
    ln_m = 2.0 * r * poly
    ef = e.astype(jnp.float32)
    return (ln_m + ef * _LN2_LO) + ef * _LN2_HI


def _insert(x, idx, t0, t1, t2, i0, i1, i2):
    c0 = x > t0
    c1 = x > t1
    c2 = x > t2
    nt0 = jnp.where(c0, x, t0)
    ni0 = jnp.where(c0, idx, i0)
    nt1 = jnp.where(c0, t0, jnp.where(c1, x, t1))
    ni1 = jnp.where(c0, i0, jnp.where(c1, idx, i1))
    nt2 = jnp.where(c1, t1, jnp.where(c2, x, t2))
    ni2 = jnp.where(c1, i1, jnp.where(c2, idx, i2))
    return nt0, nt1, nt2, ni0, ni1, ni2


def _sc_body(x_hbm, tok_hbm, sc_hbm, xv, tokv, scv):
    wid = lax.axis_index("c") * 16 + lax.axis_index("s")

    @pl.when(wid < _B)
    def _():
        b = wid
        pltpu.sync_copy(x_hbm.at[b], xv)
        lane = lax.iota(jnp.int32, 16)
        ninf = jnp.full((16,), _NEG_INF, jnp.float32)
        bigv = jnp.full((16,), _BIG, jnp.int32)
        zerov = jnp.zeros((16,), jnp.float32)

        row_m, row_s, row_v, row_t = [], [], [], []
        for r in range(_L):
            def fused_body(i, carry, r=r):
                t0, t1, t2, i0, i1, i2, sa, idx = carry
                x = xv[r, pl.ds(i * 16, 16)]
                sa = sa + jnp.exp(x)
                t0, t1, t2, i0, i1, i2 = _insert(
                    x, idx, t0, t1, t2, i0, i1, i2)
                return t0, t1, t2, i0, i1, i2, sa, idx + 16

            t0, t1, t2, i0, i1, i2, sa, _ = lax.fori_loop(
                0, _FULL, fused_body,
                (ninf, ninf, ninf, bigv, bigv, bigv, zerov, lane),
                unroll=8)

            x = xv[r, pl.ds(_TAIL, 16)]
            new = lane >= _NEW0
            sa = sa + jnp.where(new, jnp.exp(x), 0.0)
            t0, t1, t2, i0, i1, i2 = _insert(
                jnp.where(new, x, _NEG_INF), _TAIL + lane,
                t0, t1, t2, i0, i1, i2)

            vs = [t0, t1, t2]
            ids = [i0, i1, i2]
            vals_r, toks_r = [], []
            for _round in range(_K):
                mv = jnp.maximum(jnp.maximum(vs[0], vs[1]), vs[2])
                m_sc = jnp.max(mv)
                cand = jnp.full((16,), _BIG, jnp.int32)
                for j in range(_K):
                    cand = jnp.minimum(
                        cand, jnp.where(vs[j] == m_sc, ids[j], _BIG))
                idx_sc = jnp.min(cand)
                vals_r.append(m_sc)
                toks_r.append(idx_sc)
                for j in range(_K):
                    hit = (vs[j] == m_sc) & (ids[j] == idx_sc)
                    vs[j] = jnp.where(hit, _NEG_INF, vs[j])
            m_r = vals_r[0]
            row_m.append(m_r)
            row_s.append(jnp.sum(sa))
            row_v.append(vals_r)
            row_t.append(toks_r)

        rowid = jnp.where(lane < 3, 0,
                          jnp.where(lane < 6, 1, jnp.where(lane < 9, 2, 3)))
        m_vec = jnp.where(rowid == 0, row_m[0],
                          jnp.where(rowid == 1, row_m[1],
                                    jnp.where(rowid == 2, row_m[2], row_m[3])))
        s_vec = jnp.where(rowid == 0, row_s[0],
                          jnp.where(rowid == 1, row_s[1],
                                    jnp.where(rowid == 2, row_s[2], row_s[3])))
        s_vec = s_vec * jnp.exp(-m_vec)
        vraw = jnp.zeros((16,), jnp.float32)
        for r in range(_L):
            for i in range(_K):
                vraw = jnp.where(lane == r * _K + i, row_v[r][i], vraw)
        p_vec = jnp.exp(vraw - m_vec) / s_vec
        lp_vec = _poly_log(p_vec + _EPS)
        lp = [[jnp.sum(jnp.where(lane == r * _K + i, lp_vec, 0.0))
               for i in range(_K)] for r in range(_L)]

        scores = [lp[0][i] for i in range(_K)]
        seqs = [[row_t[0][k], 0, 0, 0] for k in range(_K)]
        for t in range(1, _L):
            c9 = [scores[k] + lp[t][i] for k in range(_K) for i in range(_K)]
            new_scores, new_seqs = [], []
            for _j in range(_K):
                best = c9[0]
                for q in range(1, 9):
                    best = jnp.maximum(best, c9[q])
                beam = 2
                ipick = _K - 1
                for q in range(8, -1, -1):
                    beam = jnp.where(c9[q] == best, q // _K, beam)
                    ipick = jnp.where(c9[q] == best, q % _K, ipick)
                tok = jnp.where(ipick == 0, row_t[t][0],
                                jnp.where(ipick == 1, row_t[t][1],
                                          row_t[t][2]))
                g = []
                for s in range(_L):
                    g.append(jnp.where(beam == 0, seqs[0][s],
                                       jnp.where(beam == 1, seqs[1][s],
                                                 seqs[2][s])))
                g[t] = tok
                new_scores.append(best)
                new_seqs.append(g)
                sel = beam * _K + ipick
                c9 = [jnp.where(sel == q, _NEG_INF, c9[q]) for q in range(9)]
            scores, seqs = new_scores, new_seqs

        tokvec = jnp.zeros((16,), jnp.int32)
        for t in range(_L):
            for j in range(_K):
                tokvec = jnp.where(lane == t * 4 + j,
                                   seqs[j][t].astype(jnp.int32), tokvec)
        scvec = jnp.zeros((16,), jnp.float32)
        for j in range(_K):
            scvec = jnp.where(lane == j, scores[j], scvec)
        tokv[...] = tokvec
        scv[...] = scvec
        pltpu.sync_copy(tokv, tok_hbm.at[b])
        pltpu.sync_copy(scv, sc_hbm.at[b])


def kernel(logits):
    mesh = plsc.VectorSubcoreMesh(core_axis_name="c", subcore_axis_name="s",
                                  num_cores=2, num_subcores=16)
    f = pl.kernel(
        _sc_body,
        out_type=(
            jax.ShapeDtypeStruct((_B, 16), jnp.int32),
            jax.ShapeDtypeStruct((_B, 16), jnp.float32),
        ),
        mesh=mesh,
        scratch_types=[
            pltpu.VMEM((_L, _V), jnp.float32),
            pltpu.VMEM((16,), jnp.int32),
            pltpu.VMEM((16,), jnp.float32),
        ],
        compiler_params=pltpu.CompilerParams(needs_layout_passes=False),
    )
    tok_p, sc_p = f(logits)
    return tok_p.reshape(_B, _L, 4)[:, :, :_K], sc_p[:, :_K]


# speedup vs baseline: 1.0569x; 1.0569x over previous
import functools

import jax
import jax.numpy as jnp
from jax import lax
from jax.experimental import pallas as pl
from jax.experimental.pallas import tpu as pltpu
from jax.experimental.pallas import tpu_sc as plsc

_K = 3
_L = 4
_B = 8
_V = 1000
_FULL = _V // 16
_TAIL = _V - 16
_NEW0 = _FULL * 16 - _TAIL
_EPS = 2.220446049250313e-16
_NEG_INF = float("-inf")
_BIG = 1 << 20
_SQRT2 = 1.4142135381698608
_LN2_HI = 0.693359375
_LN2_LO = -2.12194440054690583e-4


def _poly_log(x):
    bits = lax.bitcast_convert_type(x, jnp.int32)
    e = (bits >> 23) - 127
    m = lax.bitcast_convert_type(
        (bits & 0x7FFFFF) | 0x3F800000, jnp.float32)
    big = m > _SQRT2
    m = jnp.where(big, m * 0.5, m)
    e = jnp.where(big, e + 1, e)
    r = (m - 1.0) / (m + 1.0)
    t = r * r
    poly = 1.0 + t * (1.0 / 3.0 + t * (0.2 + t * (1.0 / 7.0 + t * (1.0 / 9.0))))
    ln_m = 2.0 * r * poly
    ef = e.astype(jnp.float32)
    return (ln_m + ef * _LN2_LO) + ef * _LN2_HI


def _insert(x, idx, t0, t1, t2, i0, i1, i2):
    c0 = x > t0
    c1 = x > t1
    c2 = x > t2
    nt0 = jnp.where(c0, x, t0)
    ni0 = jnp.where(c0, idx, i0)
    nt1 = jnp.where(c0, t0, jnp.where(c1, x, t1))
    ni1 = jnp.where(c0, i0, jnp.where(c1, idx, i1))
    nt2 = jnp.where(c1, t1, jnp.where(c2, x, t2))
    ni2 = jnp.where(c1, i1, jnp.where(c2, idx, i2))
    return nt0, nt1, nt2, ni0, ni1, ni2


def _sc_body(x_hbm, tok_hbm, sc_hbm, xv, tokv, scv):
    wid = lax.axis_index("c") * 16 + lax.axis_index("s")

    @pl.when(wid < _B)
    def _():
        b = wid
        pltpu.sync_copy(x_hbm.at[b], xv)
        lane = lax.iota(jnp.int32, 16)
        ninf = jnp.full((16,), _NEG_INF, jnp.float32)
        bigv = jnp.full((16,), _BIG, jnp.int32)
        zerov = jnp.zeros((16,), jnp.float32)

        row_m, row_s, row_v, row_t = [], [], [], []
        for r in range(_L):
            def fused_body(i, carry, r=r):
                t0, t1, t2, i0, i1, i2, sa, idx = carry
                x = xv[r, pl.ds(i * 16, 16)]
                sa = sa + jnp.exp(x)
                t0, t1, t2, i0, i1, i2 = _insert(
                    x, idx, t0, t1, t2, i0, i1, i2)
                return t0, t1, t2, i0, i1, i2, sa, idx + 16

            t0, t1, t2, i0, i1, i2, sa, _ = lax.fori_loop(
                0, _FULL, fused_body,
                (ninf, ninf, ninf, bigv, bigv, bigv, zerov, lane),
                unroll=8)

            x = xv[r, pl.ds(_TAIL, 16)]
            new = lane >= _NEW0
            sa = sa + jnp.where(new, jnp.exp(x), 0.0)
            t0, t1, t2, i0, i1, i2 = _insert(
                jnp.where(new, x, _NEG_INF), _TAIL + lane,
                t0, t1, t2, i0, i1, i2)

            vs = [t0, t1, t2]
            ids = [i0, i1, i2]
            vals_r, toks_r = [], []
            for _round in range(_K):
                mv = jnp.maximum(jnp.maximum(vs[0], vs[1]), vs[2])
                m_sc = jnp.max(mv)
                cand = jnp.full((16,), _BIG, jnp.int32)
                for j in range(_K):
                    cand = jnp.minimum(
                        cand, jnp.where(vs[j] == m_sc, ids[j], _BIG))
                idx_sc = jnp.min(cand)
                vals_r.append(m_sc)
                toks_r.append(idx_sc)
                for j in range(_K):
                    hit = (vs[j] == m_sc) & (ids[j] == idx_sc)
                    vs[j] = jnp.where(hit, _NEG_INF, vs[j])
            m_r = vals_r[0]
            row_m.append(m_r)
            row_s.append(jnp.sum(sa))
            row_v.append(vals_r)
            row_t.append(toks_r)

        rowid = jnp.where(lane < 3, 0,
                          jnp.where(lane < 6, 1, jnp.where(lane < 9, 2, 3)))
        m_vec = jnp.where(rowid == 0, row_m[0],
                          jnp.where(rowid == 1, row_m[1],
                                    jnp.where(rowid == 2, row_m[2], row_m[3])))
        s_vec = jnp.where(rowid == 0, row_s[0],
                          jnp.where(rowid == 1, row_s[1],
                                    jnp.where(rowid == 2, row_s[2], row_s[3])))
        s_vec = s_vec * jnp.exp(-m_vec)
        vraw = jnp.zeros((16,), jnp.float32)
        for r in range(_L):
            for i in range(_K):
                vraw = jnp.where(lane == r * _K + i, row_v[r][i], vraw)
        p_vec = jnp.exp(vraw - m_vec) / s_vec
        lp_vec = _poly_log(p_vec + _EPS)
        lp = [[jnp.sum(jnp.where(lane == r * _K + i, lp_vec, 0.0))
               for i in range(_K)] for r in range(_L)]

        scores = [lp[0][i] for i in range(_K)]
        seqs = [[row_t[0][k], 0, 0, 0] for k in range(_K)]
        for t in range(1, _L):
            c9 = [scores[k] + lp[t][i] for k in range(_K) for i in range(_K)]
            new_scores, new_seqs = [], []
            for _j in range(_K):
                best = c9[0]
                for q in range(1, 9):
                    best = jnp.maximum(best, c9[q])
                beam = 2
                ipick = _K - 1
                for q in range(8, -1, -1):
                    beam = jnp.where(c9[q] == best, q // _K, beam)
                    ipick = jnp.where(c9[q] == best, q % _K, ipick)
                tok = jnp.where(ipick == 0, row_t[t][0],
                                jnp.where(ipick == 1, row_t[t][1],
                                          row_t[t][2]))
                g = []
                for s in range(_L):
                    g.append(jnp.where(beam == 0, seqs[0][s],
                                       jnp.where(beam == 1, seqs[1][s],
                                                 seqs[2][s])))
                g[t] = tok
                new_scores.append(best)
                new_seqs.append(g)
                sel = beam * _K + ipick
                c9 = [jnp.where(sel == q, _NEG_INF, c9[q]) for q in range(9)]
            scores, seqs = new_scores, new_seqs

        tokvec = jnp.zeros((16,), jnp.int32)
        for t in range(_L):
            for j in range(_K):
                tokvec = jnp.where(lane == t * 4 + j,
                                   seqs[j][t].astype(jnp.int32), tokvec)
        scvec = jnp.zeros((16,), jnp.float32)
        for j in range(_K):
            scvec = jnp.where(lane == j, scores[j], scvec)
        tokv[...] = tokvec
        scv[...] = scvec
        pltpu.sync_copy(tokv, tok_hbm.at[b])
        pltpu.sync_copy(scv, sc_hbm.at[b])


def kernel(logits):
    mesh = plsc.VectorSubcoreMesh(core_axis_name="c", subcore_axis_name="s",
                                  num_cores=1, num_subcores=16)
    f = pl.kernel(
        _sc_body,
        out_type=(
            jax.ShapeDtypeStruct((_B, 16), jnp.int32),
            jax.ShapeDtypeStruct((_B, 16), jnp.float32),
        ),
        mesh=mesh,
        scratch_types=[
            pltpu.VMEM((_L, _V), jnp.float32),
            pltpu.VMEM((16,), jnp.int32),
            pltpu.VMEM((16,), jnp.float32),
        ],
        compiler_params=pltpu.CompilerParams(needs_layout_passes=False),
    )
    tok_p, sc_p = f(logits)
    return tok_p.reshape(_B, _L, 4)[:, :, :_K], sc_p[:, :_K]


# speedup vs baseline: 4.5829x; 4.3362x over previous
import jax
import jax.numpy as jnp
from jax import lax
from jax.experimental import pallas as pl

_TOP_K = 3
_EPS = 2.220446049250313e-16
_NEG_INF = float("-inf")


def _beam_kernel(x_ref, tok_ref, sc_ref):
    B, L, V = x_ref.shape
    K = _TOP_K

    x = x_ref[...]
    m = jnp.max(x, axis=2, keepdims=True)
    e = jnp.exp(x - m)
    s = jnp.sum(e, axis=2, keepdims=True)
    lp = jnp.log(e / s + _EPS)

    iota_v = lax.broadcasted_iota(jnp.int32, (B, L, V), 2)
    vals, toks = [], []
    work = lp
    for i in range(K):
        v = jnp.max(work, axis=2, keepdims=True)
        idx = jnp.min(jnp.where(work == v, iota_v, V), axis=2, keepdims=True)
        vals.append(v)
        toks.append(idx)
        if i + 1 < K:
            work = jnp.where(iota_v == idx, _NEG_INF, work)

    def at_t(a, t):
        return a[:, t, :]

    scores = [at_t(vals[i], 0) for i in range(K)]
    iota_c = lax.broadcasted_iota(jnp.int32, (B, L), 1)
    seqs = [jnp.where(iota_c == 0, at_t(toks[i], 0), 0) for i in range(K)]

    for t in range(1, L):
        vt = [at_t(vals[i], t) for i in range(K)]
        tt = [at_t(toks[i], t) for i in range(K)]
        c = [scores[k] + vt[i] for k in range(K) for i in range(K)]
        new_scores, new_seqs = [], []
        for _j in range(K):
            m01 = jnp.maximum(c[0], c[1])
            m23 = jnp.maximum(c[2], c[3])
            m45 = jnp.maximum(c[4], c[5])
            m67 = jnp.maximum(c[6], c[7])
            best = jnp.maximum(
                jnp.maximum(jnp.maximum(m01, m23), jnp.maximum(m45, m67)),
                c[8])
            beam = jnp.full((B, 1), K - 1, jnp.int32)
            ipick = jnp.full((B, 1), K - 1, jnp.int32)
            for q in range(8, -1, -1):
                hit = c[q] == best
                beam = jnp.where(hit, q // K, beam)
                ipick = jnp.where(hit, q % K, ipick)
            tok = jnp.where(ipick == 0, tt[0],
                            jnp.where(ipick == 1, tt[1], tt[2]))
            g = jnp.where(beam == 0, seqs[0],
                          jnp.where(beam == 1, seqs[1], seqs[2]))
            g = jnp.where(iota_c == t, tok, g)
            new_scores.append(best)
            new_seqs.append(g)
            sel = beam * K + ipick
            c = [jnp.where(sel == q, _NEG_INF, c[q]) for q in range(9)]
        scores, seqs = new_scores, new_seqs

    tok_ref[...] = jnp.stack(seqs, axis=-1).astype(jnp.int32)
    sc_ref[...] = jnp.concatenate(scores, axis=1)


def kernel(logits):
    B, L, V = logits.shape
    return pl.pallas_call(
        _beam_kernel,
        out_shape=(
            jax.ShapeDtypeStruct((B, L, _TOP_K), jnp.int32),
            jax.ShapeDtypeStruct((B, _TOP_K), jnp.float32),
        ),
    )(logits)
